# Initial kernel scaffold; baseline (speedup 1.0000x reference)
import jax
import jax.numpy as jnp
from jax.experimental import pallas as pl


def kernel(x_seq, edge_index, edge_weight, Wg, bg, Wres, bres, Whead, bhead):
    raise NotImplementedError("write your pallas kernel here")


# speedup vs baseline: 5.7920x; 5.7920x over previous
import jax
import jax.numpy as jnp
from jax.experimental import pallas as pl
from jax.experimental.pallas import tpu as pltpu

ALPHA = 0.5


def _recurrent_kernel(x_ref, wx_ref, bx_ref, wh_ref, bh_ref, whh_ref,
                      bhh_ref, whead_ref, bhead_ref, out_ref):
    T = x_ref.shape[0]
    D = x_ref.shape[2]
    L = wx_ref.shape[0]
    f32 = jnp.float32

    h = [None] * L
    for t in range(T):
        out = x_ref[t]
        for l in range(L):
            H = h[l]
            xz = jnp.dot(out, wx_ref[l], preferred_element_type=f32)
            xz = xz + bx_ref[l]
            if H is None:
                hz = bh_ref[l]
                hh = bhh_ref[l]
                Z = jax.nn.sigmoid(xz[:, :D] + hz[:, :D])
                R = jax.nn.sigmoid(xz[:, D:2 * D] + hz[:, D:])
                H_tilde = jnp.tanh(xz[:, 2 * D:3 * D] + hh)
                H_new = (1.0 - Z) * H_tilde
            else:
                hz = jnp.dot(H, wh_ref[l], preferred_element_type=f32)
                hz = hz + bh_ref[l]
                Z = jax.nn.sigmoid(xz[:, :D] + hz[:, :D])
                R = jax.nn.sigmoid(xz[:, D:2 * D] + hz[:, D:])
                hh = jnp.dot(H * R, whh_ref[l], preferred_element_type=f32)
                H_tilde = jnp.tanh(xz[:, 2 * D:3 * D] + hh + bhh_ref[l])
                H_new = Z * H + (1.0 - Z) * H_tilde
            residual = xz[:, 3 * D:]
            h_w = jax.nn.relu((1.0 - ALPHA) * H_new + ALPHA * residual)
            h[l] = h_w
            out = h_w
    pred = jnp.dot(h[-1], whead_ref[...], preferred_element_type=f32)
    out_ref[...] = pred + bhead_ref[...]


def kernel(x_seq, edge_index, edge_weight, Wg, bg, Wres, bres, Whead, bhead):
    del edge_index, edge_weight
    T, N, D = x_seq.shape
    L = Wg.shape[0]
    HZN = Whead.shape[0]

    Wx = jnp.concatenate([Wg[:, 0], Wg[:, 2], Wg[:, 4], Wres], axis=-1)
    Wh = jnp.concatenate([Wg[:, 1], Wg[:, 3]], axis=-1)
    Whh = Wg[:, 5]
    bx = jnp.concatenate([bg[:, 0], bg[:, 2], bg[:, 4], bres],
                         axis=-1)[:, None, :]
    bh = jnp.concatenate([bg[:, 1], bg[:, 3]], axis=-1)[:, None, :]
    bhh = bg[:, 5][:, None, :]
    Whead_T = Whead.T
    bhead2 = bhead[None, :]

    BN = 1000
    grid = (N // BN,)
    rep3 = lambda i: (0, 0, 0)
    rep2 = lambda i: (0, 0)
    return pl.pallas_call(
        _recurrent_kernel,
        grid=grid,
        in_specs=[
            pl.BlockSpec((T, BN, D), lambda i: (0, i, 0)),
            pl.BlockSpec((L, D, 4 * D), rep3),
            pl.BlockSpec((L, 1, 4 * D), rep3),
            pl.BlockSpec((L, D, 2 * D), rep3),
            pl.BlockSpec((L, 1, 2 * D), rep3),
            pl.BlockSpec((L, D, D), rep3),
            pl.BlockSpec((L, 1, D), rep3),
            pl.BlockSpec((D, HZN), rep2),
            pl.BlockSpec((1, HZN), rep2),
        ],
        out_specs=pl.BlockSpec((BN, HZN), lambda i: (i, 0)),
        out_shape=jax.ShapeDtypeStruct((N, HZN), x_seq.dtype),
        compiler_params=pltpu.CompilerParams(
            dimension_semantics=("parallel",)),
    )(x_seq, Wx, bx, Wh, bh, Whh, bhh, Whead_T, bhead2)
